# baseline (speedup 1.0000x reference)
import jax
import jax.numpy as jnp
from jax import lax
from jax.experimental import pallas as pl
from jax.experimental.pallas import tpu as pltpu
from jax.experimental.pallas import tpu_sc as plsc

N = 10000
E = 320000
D = 128
NPAD = 10112
NC = 2
NS = 16
NW = NC * NS
IDXW = 64
EPAD = 327680
CPW = EPAD // IDXW // NW
PGC = 16
NPG = CPW // PGC
RPT = NPAD // NS
CT = 128


def _sc_body(NB, with_cnt, xa_hbm, edge_hbm, zeros_hbm, ident_hbm, out_hbm,
             cnt_hbm, idx_s, idx_d, rows, cnt_priv, ident, shared,
             shared_cnt, sem):
    cid = lax.axis_index("c")
    sid = lax.axis_index("s")
    wid = cid * NS + sid

    gsems = sem[:NB]
    isems = sem[NB:]

    pltpu.sync_copy(zeros_hbm, shared.at[pl.ds(sid * RPT, RPT)])
    pltpu.sync_copy(edge_hbm.at[0, wid, pl.ds(0, PGC)], idx_s.at[0])
    pltpu.sync_copy(edge_hbm.at[1, wid, pl.ds(0, PGC)], idx_d.at[0])

    if with_cnt:
        pltpu.sync_copy(ident_hbm, ident)
        zero16 = jnp.zeros((16,), jnp.float32)

        def zrow(r, carry):
            for c in range(CT // 16):
                cnt_priv[r, pl.ds(c * 16, 16)] = zero16
            return carry

        lax.fori_loop(0, CT, zrow, 0)

        @pl.when(sid == 0)
        def _():
            pltpu.sync_copy(cnt_priv, shared_cnt)

    plsc.subcore_barrier()

    def gather(pb, k, buf):
        pltpu.async_copy(xa_hbm.at[idx_s.at[pb, k]], rows.at[buf],
                         gsems[buf])

    def wait_gather(pb, k, buf):
        pltpu.make_async_copy(xa_hbm.at[idx_s.at[pb, k]], rows.at[buf],
                              gsems[buf]).wait()

    ones16 = jnp.full((16,), 1.0, jnp.float32)

    for p in range(NPG):
        pb = p % 2
        if p > 0:
            pltpu.make_async_copy(edge_hbm.at[0, wid, pl.ds(p * PGC, PGC)],
                                  idx_s.at[pb], isems[pb]).wait()
            pltpu.make_async_copy(edge_hbm.at[1, wid, pl.ds(p * PGC, PGC)],
                                  idx_d.at[pb], isems[pb]).wait()
        if p < NPG - 1:
            nb = (p + 1) % 2
            pltpu.async_copy(edge_hbm.at[0, wid, pl.ds((p + 1) * PGC, PGC)],
                             idx_s.at[nb], isems[nb])
            pltpu.async_copy(edge_hbm.at[1, wid, pl.ds((p + 1) * PGC, PGC)],
                             idx_d.at[nb], isems[nb])

        for m in range(NB):
            gather(pb, m, m)

        def step(k, carry, pb=pb):
            for m in range(NB):
                c = NB * k + m
                wait_gather(pb, c, m)
                pltpu.sync_copy(rows.at[m], shared.at[idx_d.at[pb, c]],
                                add=True)
                gather(pb, c + NB, m)
            return carry

        lax.fori_loop(0, PGC // NB - 1, step, 0)
        for m in range(NB):
            c = PGC - NB + m
            wait_gather(pb, c, m)
            pltpu.sync_copy(rows.at[m], shared.at[idx_d.at[pb, c]],
                            add=True)

        if with_cnt:
            def cstep(g, carry, pb=pb):
                idx = idx_d[pb, g >> 2, pl.ds((g & 3) * 16, 16)]
                plsc.addupdate_scatter(cnt_priv, [idx >> 7, idx & 127],
                                       ones16)
                return carry

            lax.fori_loop(0, PGC * IDXW // 16, cstep, 0)

    if with_cnt:
        plsc.subcore_barrier()
        pltpu.sync_copy(cnt_priv, shared_cnt.at[ident.at[0]], add=True)
    plsc.subcore_barrier()

    pltpu.sync_copy(shared.at[pl.ds(sid * RPT, RPT)],
                    out_hbm.at[pl.ds(cid * NPAD + sid * RPT, RPT)])
    if with_cnt:
        @pl.when(sid == 0)
        def _():
            pltpu.sync_copy(shared_cnt, cnt_hbm.at[cid])


def _make_sc_agg(with_cnt):
    mesh = plsc.VectorSubcoreMesh(core_axis_name="c", subcore_axis_name="s")
    nb = 2 if with_cnt else 4
    if with_cnt:
        out_type = [jax.ShapeDtypeStruct((2 * NPAD, D), jnp.float32),
                    jax.ShapeDtypeStruct((2, CT, CT), jnp.float32)]

        def body(xa_hbm, edge_hbm, zeros_hbm, ident_hbm, out_hbm, cnt_hbm,
                 idx_s, idx_d, rows, cnt_priv, ident, shared, shared_cnt,
                 sem):
            _sc_body(nb, True, xa_hbm, edge_hbm, zeros_hbm, ident_hbm,
                     out_hbm, cnt_hbm, idx_s, idx_d, rows, cnt_priv,
                     ident, shared, shared_cnt, sem)

        scratch = [
            pltpu.VMEM((2, PGC, IDXW), jnp.int32),
            pltpu.VMEM((2, PGC, IDXW), jnp.int32),
            pltpu.VMEM((nb, IDXW, D), jnp.float32),
            pltpu.VMEM((CT, CT), jnp.float32),
            pltpu.VMEM((1, CT), jnp.int32),
            pltpu.VMEM_SHARED((NPAD, D), jnp.float32),
            pltpu.VMEM_SHARED((CT, CT), jnp.float32),
            tuple(pltpu.SemaphoreType.DMA for _ in range(nb + 2)),
        ]
    else:
        out_type = jax.ShapeDtypeStruct((2 * NPAD, D), jnp.float32)

        def body(xa_hbm, edge_hbm, zeros_hbm, out_hbm,
                 idx_s, idx_d, rows, shared, sem):
            _sc_body(nb, False, xa_hbm, edge_hbm, zeros_hbm, None,
                     out_hbm, None, idx_s, idx_d, rows, None, None,
                     shared, None, sem)

        scratch = [
            pltpu.VMEM((2, PGC, IDXW), jnp.int32),
            pltpu.VMEM((2, PGC, IDXW), jnp.int32),
            pltpu.VMEM((nb, IDXW, D), jnp.float32),
            pltpu.VMEM_SHARED((NPAD, D), jnp.float32),
            tuple(pltpu.SemaphoreType.DMA for _ in range(nb + 2)),
        ]

    return pl.kernel(
        body,
        out_type=out_type,
        mesh=mesh,
        scratch_types=scratch,
        compiler_params=pltpu.CompilerParams(use_tc_tiling_on_sc=True,
                                             needs_layout_passes=False),
    )


def _sage_block(a0, a1, x, recip, wl, bl, wr):
    agg = a0 + a1
    mean = agg * recip
    out = (jnp.dot(mean, wl, preferred_element_type=jnp.float32) + bl
           + jnp.dot(x, wr, preferred_element_type=jnp.float32))
    nrm = jnp.sqrt(jnp.sum(out * out, axis=1, keepdims=True))
    out = out / jnp.maximum(nrm, 1e-12)
    return jnp.maximum(out, 0.0)


def _layer0_body(a0_ref, a1_ref, xa_ref, c0_ref, c1_ref, wl_ref, bl_ref,
                 wr_ref, o_ref, r_ref):
    i = pl.program_id(0)
    cnt = c0_ref[pl.ds(i, 1), :] + c1_ref[pl.ds(i, 1), :]
    recip_row = 1.0 / jnp.maximum(cnt, 1.0)
    recip = recip_row.T
    r_ref[...] = jnp.broadcast_to(recip, (o_ref.shape[0], 8))
    o_ref[...] = _sage_block(a0_ref[...], a1_ref[...], xa_ref[...],
                             recip, wl_ref[...], bl_ref[...], wr_ref[...])


def _layer_body(a0_ref, a1_ref, xa_ref, r_ref, wl_ref, bl_ref, wr_ref,
                o_ref):
    o_ref[...] = _sage_block(a0_ref[...], a1_ref[...], xa_ref[...],
                             r_ref[:, :1], wl_ref[...], bl_ref[...],
                             wr_ref[...])


def _head_body(a0_ref, a1_ref, xa_ref, r_ref, wl_ref, bl_ref, wr_ref,
               w0_ref, b0_ref, w1_ref, b1_ref, o_ref):
    x3 = _sage_block(a0_ref[...], a1_ref[...], xa_ref[...], r_ref[:, :1],
                     wl_ref[...], bl_ref[...], wr_ref[...])
    h = jnp.maximum(jnp.dot(x3, w0_ref[...],
                            preferred_element_type=jnp.float32)
                    + b0_ref[...], 0.0)
    o_ref[...] = (jnp.dot(h, w1_ref[...], preferred_element_type=jnp.float32)
                  + b1_ref[...])


_BR = 1264


def _row_spec(w, off=0):
    return pl.BlockSpec((_BR, w), lambda i, o=off: (i + o, 0))


def _full_spec(a, b):
    return pl.BlockSpec((a, b), lambda i: (0, 0))


def _make_tc_layer0(interpret=False):
    return pl.pallas_call(
        _layer0_body,
        grid=(NPAD // _BR,),
        in_specs=[
            _row_spec(D), _row_spec(D, NPAD // _BR), _row_spec(D),
            _full_spec(NPAD // _BR, _BR),
            _full_spec(NPAD // _BR, _BR),
            _full_spec(D, D), _full_spec(1, D), _full_spec(D, D),
        ],
        out_specs=[_row_spec(D), _row_spec(8)],
        out_shape=[jax.ShapeDtypeStruct((NPAD, D), jnp.float32),
                   jax.ShapeDtypeStruct((NPAD, 8), jnp.float32)],
        interpret=interpret,
    )


def _make_tc_layer(interpret=False):
    return pl.pallas_call(
        _layer_body,
        grid=(NPAD // _BR,),
        in_specs=[
            _row_spec(D), _row_spec(D, NPAD // _BR), _row_spec(D),
            _row_spec(8),
            _full_spec(D, D), _full_spec(1, D), _full_spec(D, D),
        ],
        out_specs=_row_spec(D),
        out_shape=jax.ShapeDtypeStruct((NPAD, D), jnp.float32),
        interpret=interpret,
    )


def _make_tc_head(interpret=False):
    return pl.pallas_call(
        _head_body,
        grid=(NPAD // _BR,),
        in_specs=[
            _row_spec(D), _row_spec(D, NPAD // _BR), _row_spec(D),
            _row_spec(8),
            _full_spec(D, D), _full_spec(1, D), _full_spec(D, D),
            _full_spec(D, D), _full_spec(1, D),
            _full_spec(D, D), _full_spec(1, D),
        ],
        out_specs=_row_spec(D),
        out_shape=jax.ShapeDtypeStruct((NPAD, D), jnp.float32),
        interpret=interpret,
    )


def kernel(x, edge_index, Wl0, bl0, Wr0, Wl1, bl1, Wr1, Wl2, bl2, Wr2,
           Wlin0, blin0, Wlin1, blin1):
    pad_e = jnp.broadcast_to(jnp.array([[0], [N]], jnp.int32),
                             (2, EPAD - E))
    e4d = jnp.concatenate([edge_index, pad_e], axis=1).reshape(
        2, NW, CPW, IDXW)
    xa = jnp.zeros((NPAD, D), jnp.float32).at[:N].set(x)
    zeros_stage = jnp.zeros((RPT, D), jnp.float32)
    ident = jnp.arange(CT, dtype=jnp.int32).reshape(1, CT)

    out_dim = Wlin1.shape[1]
    w1p = jnp.zeros((D, D), jnp.float32).at[:, :out_dim].set(Wlin1)
    b1p = jnp.zeros((1, D), jnp.float32).at[0, :out_dim].set(blin1)

    sc_cnt = _make_sc_agg(True)
    sc_agg = _make_sc_agg(False)
    tc_layer0 = _make_tc_layer0()
    tc_layer = _make_tc_layer()
    tc_head = _make_tc_head()

    partials, cntp = sc_cnt(xa, e4d, zeros_stage, ident)
    cflat = cntp.reshape(2, CT * CT)[:, :NPAD].reshape(
        2, NPAD // _BR, _BR)
    xa, recip = tc_layer0(partials, partials, xa, cflat[0], cflat[1],
                          Wl0, bl0.reshape(1, D), Wr0)

    partials = sc_agg(xa, e4d, zeros_stage)
    xa = tc_layer(partials, partials, xa, recip,
                  Wl1, bl1.reshape(1, D), Wr1)

    partials = sc_agg(xa, e4d, zeros_stage)
    out = tc_head(partials, partials, xa, recip,
                  Wl2, bl2.reshape(1, D), Wr2,
                  Wlin0, blin0.reshape(1, D), w1p, b1p)
    return out[:N, :out_dim]


# speedup vs baseline: 2.5458x; 2.5458x over previous
import functools

import jax
import jax.numpy as jnp
from jax import lax
from jax.experimental import pallas as pl
from jax.experimental.pallas import tpu as pltpu
from jax.experimental.pallas import tpu_sc as plsc

N = 10000
E = 320000
D = 128
W = 144
NPAD = 10240
NC = 2
NS = 16
NW = NC * NS
IDXW = 50
CPW = (E // IDXW) // NW
PGC = 40
NPG = CPW // PGC
NB = 4
RPT = NPAD // NS


def _sc_agg_body(xa_hbm, edge_hbm, zeros_hbm, out_hbm,
                 idx_s, idx_d, rows, shared, sem):
    cid = lax.axis_index("c")
    sid = lax.axis_index("s")
    wid = cid * NS + sid

    gsems = sem[:NB]
    isems = sem[NB:]

    pltpu.sync_copy(zeros_hbm, shared.at[pl.ds(sid * RPT, RPT)])
    pltpu.sync_copy(edge_hbm.at[0, wid, pl.ds(0, PGC)], idx_s.at[0])
    pltpu.sync_copy(edge_hbm.at[1, wid, pl.ds(0, PGC)], idx_d.at[0])
    plsc.subcore_barrier()

    def gather(pb, k, buf):
        pltpu.async_copy(xa_hbm.at[idx_s.at[pb, k]], rows.at[buf],
                         gsems[buf])

    def wait_gather(pb, k, buf):
        pltpu.make_async_copy(xa_hbm.at[idx_s.at[pb, k]], rows.at[buf],
                              gsems[buf]).wait()

    for p in range(NPG):
        pb = p % 2
        if p > 0:
            pltpu.make_async_copy(edge_hbm.at[0, wid, pl.ds(p * PGC, PGC)],
                                  idx_s.at[pb], isems[pb]).wait()
            pltpu.make_async_copy(edge_hbm.at[1, wid, pl.ds(p * PGC, PGC)],
                                  idx_d.at[pb], isems[pb]).wait()
        if p < NPG - 1:
            nb = (p + 1) % 2
            pltpu.async_copy(edge_hbm.at[0, wid, pl.ds((p + 1) * PGC, PGC)],
                             idx_s.at[nb], isems[nb])
            pltpu.async_copy(edge_hbm.at[1, wid, pl.ds((p + 1) * PGC, PGC)],
                             idx_d.at[nb], isems[nb])

        for m in range(NB):
            gather(pb, m, m)

        def step(k, carry, pb=pb):
            for m in range(NB):
                c = NB * k + m
                wait_gather(pb, c, m)
                pltpu.sync_copy(rows.at[m], shared.at[idx_d.at[pb, c]],
                                add=True)
                gather(pb, c + NB, m)
            return carry

        lax.fori_loop(0, PGC // NB - 1, step, 0)
        for m in range(NB):
            c = PGC - NB + m
            wait_gather(pb, c, m)
            pltpu.sync_copy(rows.at[m], shared.at[idx_d.at[pb, c]],
                            add=True)
    plsc.subcore_barrier()

    pltpu.sync_copy(shared.at[pl.ds(sid * RPT, RPT)],
                    out_hbm.at[pl.ds(cid * NPAD + sid * RPT, RPT)])


def _make_sc_agg():
    mesh = plsc.VectorSubcoreMesh(core_axis_name="c", subcore_axis_name="s")
    return pl.kernel(
        _sc_agg_body,
        out_type=jax.ShapeDtypeStruct((2 * NPAD, W), jnp.float32),
        mesh=mesh,
        scratch_types=[
            pltpu.VMEM((2, PGC, IDXW), jnp.int32),
            pltpu.VMEM((2, PGC, IDXW), jnp.int32),
            pltpu.VMEM((NB, IDXW, W), jnp.float32),
            pltpu.VMEM_SHARED((NPAD, W), jnp.float32),
            tuple(pltpu.SemaphoreType.DMA for _ in range(NB + 2)),
        ],
        compiler_params=pltpu.CompilerParams(use_tc_tiling_on_sc=False),
    )


def _sage_block(a0, a1, xa, wl, bl, wr):
    agg = a0[:, :D] + a1[:, :D]
    cnt = a0[:, D:D + 1] + a1[:, D:D + 1]
    x = xa[:, :D]
    mean = agg / jnp.maximum(cnt, 1.0)
    out = (jnp.dot(mean, wl, preferred_element_type=jnp.float32) + bl
           + jnp.dot(x, wr, preferred_element_type=jnp.float32))
    nrm = jnp.sqrt(jnp.sum(out * out, axis=1, keepdims=True))
    out = out / jnp.maximum(nrm, 1e-12)
    return jnp.maximum(out, 0.0)


def _layer_body(a0_ref, a1_ref, xa_ref, wl_ref, bl_ref, wr_ref, o_ref):
    out = _sage_block(a0_ref[...], a1_ref[...], xa_ref[...],
                      wl_ref[...], bl_ref[...], wr_ref[...])
    r = out.shape[0]
    o_ref[:, :D] = out
    col = lax.broadcasted_iota(jnp.int32, (r, W - D), 1)
    o_ref[:, D:W] = jnp.where(col == 0, 1.0, 0.0)


def _head_body(a0_ref, a1_ref, xa_ref, wl_ref, bl_ref, wr_ref,
               w0_ref, b0_ref, w1_ref, b1_ref, o_ref):
    x3 = _sage_block(a0_ref[...], a1_ref[...], xa_ref[...],
                     wl_ref[...], bl_ref[...], wr_ref[...])
    h = jnp.maximum(jnp.dot(x3, w0_ref[...],
                            preferred_element_type=jnp.float32)
                    + b0_ref[...], 0.0)
    o_ref[...] = (jnp.dot(h, w1_ref[...], preferred_element_type=jnp.float32)
                  + b1_ref[...])


_BR = 1280


def _row_spec(w, off=0):
    return pl.BlockSpec((_BR, w), lambda i, o=off: (i + o, 0))


def _full_spec(a, b):
    return pl.BlockSpec((a, b), lambda i: (0, 0))


def _make_tc_layer(interpret=False):
    return pl.pallas_call(
        _layer_body,
        grid=(NPAD // _BR,),
        in_specs=[
            _row_spec(W), _row_spec(W, NPAD // _BR), _row_spec(W),
            _full_spec(D, D), _full_spec(1, D), _full_spec(D, D),
        ],
        out_specs=_row_spec(W),
        out_shape=jax.ShapeDtypeStruct((NPAD, W), jnp.float32),
        interpret=interpret,
    )


def _make_tc_head(interpret=False):
    return pl.pallas_call(
        _head_body,
        grid=(NPAD // _BR,),
        in_specs=[
            _row_spec(W), _row_spec(W, NPAD // _BR), _row_spec(W),
            _full_spec(D, D), _full_spec(1, D), _full_spec(D, D),
            _full_spec(D, D), _full_spec(1, D),
            _full_spec(D, D), _full_spec(1, D),
        ],
        out_specs=_row_spec(D),
        out_shape=jax.ShapeDtypeStruct((NPAD, D), jnp.float32),
        interpret=interpret,
    )


def kernel(x, edge_index, Wl0, bl0, Wr0, Wl1, bl1, Wr1, Wl2, bl2, Wr2,
           Wlin0, blin0, Wlin1, blin1):
    xa = jnp.zeros((NPAD, W), jnp.float32)
    xa = xa.at[:N, :D].set(x)
    xa = xa.at[:N, D].set(1.0)
    e4d = edge_index.reshape(2, NW, CPW, IDXW)
    zeros_stage = jnp.zeros((RPT, W), jnp.float32)

    out_dim = Wlin1.shape[1]
    w1p = jnp.zeros((D, D), jnp.float32).at[:, :out_dim].set(Wlin1)
    b1p = jnp.zeros((1, D), jnp.float32).at[0, :out_dim].set(blin1)

    sc_agg = _make_sc_agg()
    tc_layer = _make_tc_layer()
    tc_head = _make_tc_head()

    layers = [(Wl0, bl0.reshape(1, D), Wr0),
              (Wl1, bl1.reshape(1, D), Wr1),
              (Wl2, bl2.reshape(1, D), Wr2)]

    for i, (wl, bl, wr) in enumerate(layers):
        partials = sc_agg(xa, e4d, zeros_stage)
        if i < 2:
            xa = tc_layer(partials, partials, xa, wl, bl, wr)
        else:
            out = tc_head(partials, partials, xa, wl, bl, wr,
                          Wlin0, blin0.reshape(1, D), w1p, b1p)
    return out[:N, :out_dim]
